# baseline (speedup 1.0000x reference)
import jax
import jax.numpy as jnp
from jax.experimental import pallas as pl
from jax.experimental.pallas import tpu as pltpu

_BATCH_BLK = 512


def _add_kernel(inds_ref, x_ref, pe_ref, o_ref):
    del inds_ref
    o_ref[...] = x_ref[...] + pe_ref[...]


def kernel(x, pe, inds):
    batch, seq, d_model = x.shape
    inds32 = inds.astype(jnp.int32)
    xt = jnp.transpose(x, (1, 0, 2))
    pe3 = pe.reshape(pe.shape[0], 1, d_model)

    grid = (seq, batch // _BATCH_BLK)
    out_t = pl.pallas_call(
        _add_kernel,
        grid_spec=pltpu.PrefetchScalarGridSpec(
            num_scalar_prefetch=1,
            grid=grid,
            in_specs=[
                pl.BlockSpec((1, _BATCH_BLK, d_model), lambda j, i, inds_ref: (j, i, 0)),
                pl.BlockSpec((1, 1, d_model), lambda j, i, inds_ref: (inds_ref[j], 0, 0)),
            ],
            out_specs=pl.BlockSpec((1, _BATCH_BLK, d_model), lambda j, i, inds_ref: (j, i, 0)),
        ),
        out_shape=jax.ShapeDtypeStruct((seq, batch, d_model), jnp.float32),
        compiler_params=pltpu.CompilerParams(
            dimension_semantics=("arbitrary", "arbitrary"),
        ),
    )(inds32, xt, pe3)
    return jnp.transpose(out_t, (1, 0, 2))


# speedup vs baseline: 1.0947x; 1.0947x over previous
import jax
import jax.numpy as jnp
from jax.experimental import pallas as pl
from jax.experimental.pallas import tpu as pltpu

_BATCH_BLK = 2048


def _add_kernel(inds_ref, x_ref, pe_ref, o_ref):
    del inds_ref
    o_ref[...] = x_ref[...] + pe_ref[...]


def kernel(x, pe, inds):
    batch, seq, d_model = x.shape
    inds32 = inds.astype(jnp.int32)
    xt = jnp.transpose(x, (1, 0, 2))
    pe3 = pe.reshape(pe.shape[0], 1, d_model)

    grid = (seq, batch // _BATCH_BLK)
    out_t = pl.pallas_call(
        _add_kernel,
        grid_spec=pltpu.PrefetchScalarGridSpec(
            num_scalar_prefetch=1,
            grid=grid,
            in_specs=[
                pl.BlockSpec((1, _BATCH_BLK, d_model), lambda j, i, inds_ref: (j, i, 0)),
                pl.BlockSpec((1, 1, d_model), lambda j, i, inds_ref: (inds_ref[j], 0, 0)),
            ],
            out_specs=pl.BlockSpec((1, _BATCH_BLK, d_model), lambda j, i, inds_ref: (j, i, 0)),
        ),
        out_shape=jax.ShapeDtypeStruct((seq, batch, d_model), jnp.float32),
        compiler_params=pltpu.CompilerParams(
            dimension_semantics=("arbitrary", "arbitrary"),
        ),
    )(inds32, xt, pe3)
    return jnp.transpose(out_t, (1, 0, 2))
